# baseline (speedup 1.0000x reference)
import functools

import jax
import jax.numpy as jnp
from jax.experimental import pallas as pl
from jax.experimental.pallas import tpu as pltpu
from jax.experimental.pallas import tpu_sc as plsc

T = 4096
H = 2048
E = 8
O2 = 4096

ESC = 2
ETC = E - ESC

OC = 4
CH = O2 // OC
NSTEPS = ETC * OC
NMM = 16
TCH = T // NMM

NC = 2
NS = 16
NW = NC * NS
LANES = 16
SCROWS = ESC * H
RPW = SCROWS // NW
CROW = 8
NCH = RPW // CROW
VPR = O2 // LANES
UNR = 8


def _sc_body(wf_hbm, out_hbm, buf0, buf1, res_v, sem0, sem1):
    c = jax.lax.axis_index("c")
    s = jax.lax.axis_index("s")
    wid = s * NC + c
    base = wid * RPW
    bufs = (buf0, buf1)
    sems = (sem0, sem1)
    cps = [pltpu.async_copy(wf_hbm.at[pl.ds(base, CROW), :], bufs[0], sems[0]),
           None]
    for ci in range(NCH):
        if ci + 1 < NCH:
            nb = (ci + 1) % 2
            cps[nb] = pltpu.async_copy(
                wf_hbm.at[pl.ds(base + (ci + 1) * CROW, CROW), :],
                bufs[nb], sems[nb])
        cps[ci % 2].wait()
        buf = bufs[ci % 2]
        for r in range(CROW):
            def vec_body(i, accs, buf=buf, r=r):
                vb = i * UNR
                return tuple(
                    accs[j] + buf[r, pl.ds((vb + j) * LANES, LANES)]
                    for j in range(UNR))
            accs = jax.lax.fori_loop(
                0, VPR // UNR, vec_body,
                tuple(jnp.zeros((LANES,), jnp.float32) for _ in range(UNR)))
            tot = ((accs[0] + accs[1]) + (accs[2] + accs[3])) + \
                  ((accs[4] + accs[5]) + (accs[6] + accs[7]))
            res_v[ci * CROW + r, :] = tot
    pltpu.sync_copy(res_v, out_hbm.at[pl.ds(base, RPW), :])


_sc_mesh = plsc.VectorSubcoreMesh(core_axis_name="c", subcore_axis_name="s")

_sc_ws = functools.partial(
    pl.kernel,
    out_type=jax.ShapeDtypeStruct((SCROWS, LANES), jnp.float32),
    mesh=_sc_mesh,
    scratch_types=[
        pltpu.VMEM((CROW, O2), jnp.float32),
        pltpu.VMEM((CROW, O2), jnp.float32),
        pltpu.VMEM((RPW, LANES), jnp.float32),
        pltpu.SemaphoreType.DMA,
        pltpu.SemaphoreType.DMA,
    ],
)(_sc_body)


def _tc_body(sp_ref, hid_ref, w_ref, out_ref, sh_out_ref, sh_ref, ws_ref):
    e = pl.program_id(0)
    oc = pl.program_id(1)
    k = e * OC + oc

    @pl.when(k == 0)
    def _init():
        sh_ref[...] = jnp.zeros_like(sh_ref)
        ws_ref[...] = jnp.zeros_like(ws_ref)

    @pl.when(k < NMM)
    def _mm():
        sp_c = sp_ref[:, pl.ds(k * TCH, TCH)]
        hid_c = hid_ref[pl.ds(k * TCH, TCH), :]
        sh_ref[...] += jax.lax.dot_general(
            sp_c, hid_c, (((1,), (0,)), ((), ())),
            precision=jax.lax.Precision.HIGHEST,
            preferred_element_type=jnp.float32)

    ws_ref[pl.ds(e, 1), :] += jnp.sum(w_ref[0], axis=-1)[None, :]

    @pl.when(k == NSTEPS - 1)
    def _fin():
        sh_out_ref[...] = sh_ref[...]
        out_ref[...] = jnp.sum(
            sh_ref[ESC:, :] * ws_ref[:ETC, :]).reshape(1, 1)


def _fin_body(part_ref, shf_ref, wssc_ref, out_ref):
    contrib = jnp.sum(wssc_ref[...] * shf_ref[...])
    out_ref[...] = part_ref[...] + contrib.reshape(1, 1)


def kernel(hidden_4d, sparsity, gate_up_proj):
    hidden = hidden_4d.reshape(T, H)
    sp = sparsity.reshape(E, T)
    w = gate_up_proj.reshape(E, H, O2)
    wf_sc = w[:ESC].reshape(SCROWS, O2)

    ws_sc = _sc_ws(wf_sc)

    part, sh = pl.pallas_call(
        _tc_body,
        grid=(ETC, OC),
        in_specs=[
            pl.BlockSpec((E, T), lambda e, oc: (0, 0)),
            pl.BlockSpec((T, H), lambda e, oc: (0, 0)),
            pl.BlockSpec((1, H, CH), lambda e, oc: (e + ESC, 0, oc)),
        ],
        out_specs=[pl.BlockSpec((1, 1), lambda e, oc: (0, 0)),
                   pl.BlockSpec((E, H), lambda e, oc: (0, 0))],
        out_shape=[jax.ShapeDtypeStruct((1, 1), jnp.float32),
                   jax.ShapeDtypeStruct((E, H), jnp.float32)],
        scratch_shapes=[pltpu.VMEM((E, H), jnp.float32),
                        pltpu.VMEM((ETC, H), jnp.float32)],
    )(sp, hidden, w)

    sh_sc_flat = sh[:ESC].reshape(SCROWS, 1)

    out = pl.pallas_call(
        _fin_body,
        in_specs=[pl.BlockSpec((1, 1), lambda: (0, 0)),
                  pl.BlockSpec((SCROWS, 1), lambda: (0, 0)),
                  pl.BlockSpec((SCROWS, LANES), lambda: (0, 0))],
        out_specs=pl.BlockSpec((1, 1), lambda: (0, 0)),
        out_shape=jax.ShapeDtypeStruct((1, 1), jnp.float32),
    )(part, sh_sc_flat, ws_sc)
    return out[0, 0]


# speedup vs baseline: 1.3500x; 1.3500x over previous
import functools

import jax
import jax.numpy as jnp
from jax.experimental import pallas as pl
from jax.experimental.pallas import tpu as pltpu
from jax.experimental.pallas import tpu_sc as plsc

T = 4096
H = 2048
E = 8
O2 = 4096

ESC = 2
ETC = E - ESC

OC = 4
CH = O2 // OC
NSTEPS = ETC * OC
NMM = 16
TCH = T // NMM

NC = 2
NS = 16
NW = NC * NS
LANES = 16
SCROWS = ESC * H
RPW = SCROWS // NW
CROW = 8
NCH = RPW // CROW
VPR = O2 // LANES
UNR = 8


def _sc_body(wf_hbm, out_hbm, buf0, buf1, res_v, sem0, sem1):
    c = jax.lax.axis_index("c")
    s = jax.lax.axis_index("s")
    wid = s * NC + c
    base = wid * RPW
    bufs = (buf0, buf1)
    sems = (sem0, sem1)
    cps = [pltpu.async_copy(wf_hbm.at[pl.ds(base, CROW), :], bufs[0], sems[0]),
           None]
    for ci in range(NCH):
        if ci + 1 < NCH:
            nb = (ci + 1) % 2
            cps[nb] = pltpu.async_copy(
                wf_hbm.at[pl.ds(base + (ci + 1) * CROW, CROW), :],
                bufs[nb], sems[nb])
        cps[ci % 2].wait()
        buf = bufs[ci % 2]
        for r in range(CROW):
            def vec_body(i, accs, buf=buf, r=r):
                vb = i * UNR
                return tuple(
                    accs[j] + buf[r, pl.ds((vb + j) * LANES, LANES)]
                    for j in range(UNR))
            accs = jax.lax.fori_loop(
                0, VPR // UNR, vec_body,
                tuple(jnp.zeros((LANES,), jnp.float32) for _ in range(UNR)))
            tot = ((accs[0] + accs[1]) + (accs[2] + accs[3])) + \
                  ((accs[4] + accs[5]) + (accs[6] + accs[7]))
            res_v[ci * CROW + r, :] = tot
    pltpu.sync_copy(res_v, out_hbm.at[pl.ds(base, RPW), :])


_sc_mesh = plsc.VectorSubcoreMesh(core_axis_name="c", subcore_axis_name="s")

_sc_ws = functools.partial(
    pl.kernel,
    out_type=jax.ShapeDtypeStruct((SCROWS, LANES), jnp.float32),
    mesh=_sc_mesh,
    scratch_types=[
        pltpu.VMEM((CROW, O2), jnp.float32),
        pltpu.VMEM((CROW, O2), jnp.float32),
        pltpu.VMEM((RPW, LANES), jnp.float32),
        pltpu.SemaphoreType.DMA,
        pltpu.SemaphoreType.DMA,
    ],
)(_sc_body)


def _tc_body(sp_ref, hid_ref, w_ref, out_ref, sh_out_ref, sh_ref, ws_ref):
    e = pl.program_id(0)
    oc = pl.program_id(1)
    k = e * OC + oc

    @pl.when(k == 0)
    def _init():
        sh_ref[...] = jnp.zeros_like(sh_ref)
        ws_ref[...] = jnp.zeros_like(ws_ref)

    @pl.when(k < NMM)
    def _mm():
        sp_c = sp_ref[:, pl.ds(k * TCH, TCH)]
        hid_c = hid_ref[pl.ds(k * TCH, TCH), :]
        sh_ref[...] += jax.lax.dot_general(
            sp_c, hid_c, (((1,), (0,)), ((), ())),
            precision=jax.lax.Precision.HIGHEST,
            preferred_element_type=jnp.float32)

    ws_ref[pl.ds(e, 1), :] += jnp.sum(w_ref[0], axis=-1)[None, :]

    @pl.when(k == NSTEPS - 1)
    def _fin():
        sh_out_ref[...] = sh_ref[...]
        out_ref[...] = jnp.sum(
            sh_ref[ESC:, :] * ws_ref[:ETC, :]).reshape(1, 1)


def _fin_body(part_ref, shf_ref, wssc_ref, out_ref):
    contrib = jnp.sum(wssc_ref[...] * shf_ref[...])
    out_ref[...] = part_ref[...] + contrib.reshape(1, 1)


def kernel(hidden_4d, sparsity, gate_up_proj):
    hidden = hidden_4d.reshape(T, H)
    sp = sparsity.reshape(E, T)
    w = gate_up_proj.reshape(E, H, O2)
    wf = gate_up_proj.reshape(E * H, O2)

    ws_sc = _sc_ws(wf)

    part, sh = pl.pallas_call(
        _tc_body,
        grid=(ETC, OC),
        in_specs=[
            pl.BlockSpec((E, T), lambda e, oc: (0, 0)),
            pl.BlockSpec((T, H), lambda e, oc: (0, 0)),
            pl.BlockSpec((1, H, CH), lambda e, oc: (e + ESC, 0, oc)),
        ],
        out_specs=[pl.BlockSpec((1, 1), lambda e, oc: (0, 0)),
                   pl.BlockSpec((E, H), lambda e, oc: (0, 0))],
        out_shape=[jax.ShapeDtypeStruct((1, 1), jnp.float32),
                   jax.ShapeDtypeStruct((E, H), jnp.float32)],
        scratch_shapes=[pltpu.VMEM((E, H), jnp.float32),
                        pltpu.VMEM((ETC, H), jnp.float32)],
    )(sp, hidden, w)

    sh_sc_flat = sh[:ESC].reshape(SCROWS, 1)

    out = pl.pallas_call(
        _fin_body,
        in_specs=[pl.BlockSpec((1, 1), lambda: (0, 0)),
                  pl.BlockSpec((SCROWS, 1), lambda: (0, 0)),
                  pl.BlockSpec((SCROWS, LANES), lambda: (0, 0))],
        out_specs=pl.BlockSpec((1, 1), lambda: (0, 0)),
        out_shape=jax.ShapeDtypeStruct((1, 1), jnp.float32),
    )(part, sh_sc_flat, ws_sc)
    return out[0, 0]
